# baseline (speedup 1.0000x reference)
import jax
import jax.numpy as jnp
from jax.experimental import pallas as pl
from jax.experimental.pallas import tpu as pltpu

_C = 100000
_IGNORE = 0
_SMOOTH = 0.1
_CONF = 1.0 - _SMOOTH
_EPS = _SMOOTH / (_C - 1)
_BR = 32


def _block_reduce(t, blk):
    col = jax.lax.broadcasted_iota(jnp.int32, (_BR, _C), 1)
    s = jnp.sum(blk, axis=1, keepdims=True)
    g = jnp.sum(jnp.where(col == t, blk, 0.0), axis=1, keepdims=True)
    per_row = _EPS * s + (_CONF - _EPS) * g
    return jnp.sum(jnp.where(t != _IGNORE, per_row, 0.0))


def _body(tgt0_ref, tgt1_ref, logit0_ref, logit1_ref, out_ref):
    p0 = _block_reduce(tgt0_ref[...], logit0_ref[...])
    p1 = _block_reduce(tgt1_ref[...], logit1_ref[...])
    out_ref[...] = jnp.stack([-p0, -p1]).reshape(2, 1, 1)


def kernel(logit, target):
    n = logit.shape[0]
    tgt = target.astype(jnp.int32).reshape(n, 1)
    nblk = n // _BR
    out = pl.pallas_call(
        _body,
        grid=(nblk // 2,),
        in_specs=[
            pl.BlockSpec((_BR, 1), lambda i: (2 * i, 0)),
            pl.BlockSpec((_BR, 1), lambda i: (2 * i + 1, 0)),
            pl.BlockSpec((_BR, _C), lambda i: (2 * i, 0)),
            pl.BlockSpec((_BR, _C), lambda i: (2 * i + 1, 0)),
        ],
        out_specs=pl.BlockSpec((2, 1, 1), lambda i: (i, 0, 0)),
        out_shape=jax.ShapeDtypeStruct((nblk, 1, 1), jnp.float32),
        compiler_params=pltpu.CompilerParams(
            dimension_semantics=("arbitrary",),
        ),
    )(tgt, tgt, logit, logit)
    return jnp.sum(out)


# speedup vs baseline: 1.0032x; 1.0032x over previous
import jax
import jax.numpy as jnp
from jax.experimental import pallas as pl
from jax.experimental.pallas import tpu as pltpu

_C = 100000
_IGNORE = 0
_SMOOTH = 0.1
_CONF = 1.0 - _SMOOTH
_EPS = _SMOOTH / (_C - 1)
_BR = 32
_NBUF = 4


def _block_reduce(t, blk):
    col = jax.lax.broadcasted_iota(jnp.int32, (_BR, _C), 1)
    s = jnp.sum(blk, axis=1, keepdims=True)
    g = jnp.sum(jnp.where(col == t, blk, 0.0), axis=1, keepdims=True)
    per_row = _EPS * s + (_CONF - _EPS) * g
    return jnp.sum(jnp.where(t != _IGNORE, per_row, 0.0))


def _make_body(nblk):
    def _body(tgt_ref, logit_hbm, out_ref, b0, b1, b2, b3, s0, s1, s2, s3):
        i = pl.program_id(0)
        bufs = (b0, b1, b2, b3)
        sems = (s0, s1, s2, s3)

        @pl.when(i == 0)
        def _prime():
            for k in range(_NBUF):
                pltpu.make_async_copy(
                    logit_hbm.at[pl.ds(k * _BR, _BR), :], bufs[k], sems[k]
                ).start()

        t = tgt_ref[pl.ds(i * _BR, _BR), :]
        for k in range(_NBUF):
            @pl.when(i % _NBUF == k)
            def _step(k=k):
                pltpu.make_async_copy(
                    logit_hbm.at[pl.ds(i * _BR, _BR), :], bufs[k], sems[k]
                ).wait()
                partial = _block_reduce(t, bufs[k][...])
                out_ref[...] = jnp.full((1, 1, 1), -partial, jnp.float32)

                @pl.when(i + _NBUF < nblk)
                def _next():
                    pltpu.make_async_copy(
                        logit_hbm.at[pl.ds((i + _NBUF) * _BR, _BR), :],
                        bufs[k], sems[k],
                    ).start()

    return _body


def kernel(logit, target):
    n = logit.shape[0]
    tgt = target.astype(jnp.int32).reshape(n, 1)
    nblk = n // _BR
    out = pl.pallas_call(
        _make_body(nblk),
        grid=(nblk,),
        in_specs=[
            pl.BlockSpec((n, 1), lambda i: (0, 0)),
            pl.BlockSpec(memory_space=pl.MemorySpace.ANY),
        ],
        out_specs=pl.BlockSpec((1, 1, 1), lambda i: (i, 0, 0)),
        out_shape=jax.ShapeDtypeStruct((nblk, 1, 1), jnp.float32),
        scratch_shapes=[pltpu.VMEM((_BR, _C), jnp.float32)] * _NBUF
        + [pltpu.SemaphoreType.DMA] * _NBUF,
        compiler_params=pltpu.CompilerParams(
            dimension_semantics=("arbitrary",),
        ),
    )(tgt, logit)
    return jnp.sum(out)
